# baseline (speedup 1.0000x reference)
import functools

import jax
import jax.numpy as jnp
from jax import lax
from jax.experimental import pallas as pl
from jax.experimental.pallas import tpu as pltpu
from jax.experimental.pallas import tpu_sc as plsc

B = 1024
TAU = 20
F = 10
V = 100000
D = 16
DM = 64
NL = 2
NT = 2
IN = DM + TAU * D * F
OUT = (TAU + 1) * DM
FD = F * D

NC = 2
NS = 16
NW = NC * NS
PPW = FD // NW
LANES = 16

TRX = IN // 8
TRE = FD // 8
TCB = B // 128


def _gather_body(fut, tbl, emb5, idx_v, row_v, buf0, buf1, sem0, sem1):
    wid = lax.axis_index("s") * NC + lax.axis_index("c")
    bufs = (buf0, buf1)
    sems = (sem0, sem1)

    for k in range(PPW):
        p = PPW * wid + k
        f = p // D
        d = p % D
        trb = p // 8
        s = p % 8
        if k == 0:
            pltpu.sync_copy(fut.at[f], idx_v)
        else:
            @pl.when(f != (p - 1) // D)
            def _():
                pltpu.sync_copy(fut.at[f], idx_v)
        pltpu.sync_copy(tbl.at[f, d], row_v)

        def grp_body(g, carry):
            for bsel in range(2):
                gp = 2 * g + bsel
                buf, sem = bufs[bsel], sems[bsel]

                @pl.when(g > 0)
                def _():
                    pltpu.make_async_copy(
                        buf, emb5.at[pl.ds(2 * gp, 2), trb, :, s], sem).wait()

                for tloc in range(2):
                    t = 2 * gp + tloc
                    for c in range(B // LANES):
                        iv = idx_v[t, pl.ds(c * LANES, LANES)]
                        buf[tloc, c // 8, pl.ds((c % 8) * LANES, LANES)] = (
                            plsc.load_gather(row_v, [iv]))
                pltpu.async_copy(
                    buf, emb5.at[pl.ds(2 * gp, 2), trb, :, s], sem)
            return carry

        lax.fori_loop(0, TAU // 4, grp_body, 0)
        for bsel in range(2):
            gp = TAU // 2 - 2 + bsel
            pltpu.make_async_copy(
                bufs[bsel], emb5.at[pl.ds(2 * gp, 2), trb, :, s],
                sems[bsel]).wait()


_gather = functools.partial(
    pl.kernel,
    mesh=plsc.VectorSubcoreMesh(
        core_axis_name="c", subcore_axis_name="s",
        num_cores=NC, num_subcores=NS),
    out_type=jax.ShapeDtypeStruct((TAU, TRE, TCB, 8, 128), jnp.float32),
    scratch_types=[
        pltpu.VMEM((TAU, B), jnp.int32),
        pltpu.VMEM((V,), jnp.float32),
        pltpu.VMEM((2, 8, 128), jnp.float32),
        pltpu.VMEM((2, 8, 128), jnp.float32),
        pltpu.SemaphoreType.DMA,
        pltpu.SemaphoreType.DMA,
    ],
    compiler_params=pltpu.CompilerParams(
        use_tc_tiling_on_sc=True, needs_layout_passes=False,
        disable_bounds_checks=True),
)(_gather_body)


def _mm_body(h_ref, e_ref, w_ref, b_ref, o_ref):
    xh = h_ref[...].reshape(8, 8, 128).reshape(DM, 128)
    xe = e_ref[...].reshape(TAU, TRE, 8, 128).reshape(IN - DM, 128)
    xm = jnp.concatenate([xh, xe], axis=0)
    acc = jnp.dot(w_ref[0], xm, preferred_element_type=jnp.float32)
    o_ref[0] = acc + b_ref[0]


_matmul = pl.pallas_call(
    _mm_body,
    grid=(NT, TCB),
    in_specs=[
        pl.BlockSpec((1, 8, 1, 8, 128), lambda t, i: (NL - 1, 0, i, 0, 0)),
        pl.BlockSpec((TAU, TRE, 1, 8, 128), lambda t, i: (0, 0, i, 0, 0)),
        pl.BlockSpec((1, OUT, IN), lambda t, i: (t, 0, 0)),
        pl.BlockSpec((1, OUT, 1), lambda t, i: (t, 0, 0)),
    ],
    out_specs=pl.BlockSpec((1, OUT, 128), lambda t, i: (t, 0, i)),
    out_shape=jax.ShapeDtypeStruct((NT, OUT, B), jnp.float32),
)


def kernel(future, hidden, tables, W, b):
    fut = jnp.transpose(future.astype(jnp.int32), (2, 1, 0))
    tbl = jnp.transpose(tables, (0, 2, 1))
    h5 = (hidden.reshape(NL, B, 8, 8)
          .transpose(0, 2, 3, 1)
          .reshape(NL, 8, 8, 8, 128)
          .transpose(0, 1, 3, 2, 4))
    emb5 = _gather(fut, tbl)
    emb_out = emb5.transpose(2, 4, 0, 1, 3).reshape(B, TAU, FD)
    gc_t = _matmul(h5, emb5, jnp.transpose(W, (0, 2, 1)),
                   b.reshape(NT, OUT, 1))
    return emb_out, jnp.transpose(gc_t, (2, 0, 1))


# speedup vs baseline: 1.0008x; 1.0008x over previous
import functools

import jax
import jax.numpy as jnp
from jax import lax
from jax.experimental import pallas as pl
from jax.experimental.pallas import tpu as pltpu
from jax.experimental.pallas import tpu_sc as plsc

B = 1024
TAU = 20
F = 10
V = 100000
D = 16
DM = 64
NL = 2
NT = 2
IN = DM + TAU * D * F
OUT = (TAU + 1) * DM
FD = F * D

NC = 2
NS = 16
NW = NC * NS
PPW = FD // NW
LANES = 16

TRX = IN // 8
TRE = FD // 8
TCB = B // 128


def _gather_body(fut, tbl, emb5, idx_v, row_v, buf0, buf1, sem0, sem1):
    wid = lax.axis_index("s") * NC + lax.axis_index("c")
    bufs = (buf0, buf1)
    sems = (sem0, sem1)

    for k in range(PPW):
        p = PPW * wid + k
        f = p // D
        d = p % D
        trb = p // 8
        s = p % 8
        if k == 0:
            pltpu.sync_copy(fut.at[f], idx_v)
        else:
            @pl.when(f != (p - 1) // D)
            def _():
                pltpu.sync_copy(fut.at[f], idx_v)
        pltpu.sync_copy(tbl.at[f, d], row_v)

        def grp_body(g, carry):
            for bsel in range(2):
                gp = 2 * g + bsel
                buf, sem = bufs[bsel], sems[bsel]

                @pl.when(g > 0)
                def _():
                    pltpu.make_async_copy(
                        buf, emb5.at[pl.ds(2 * gp, 2), trb, :, s], sem).wait()

                for tloc in range(2):
                    t = 2 * gp + tloc
                    for c in range(B // LANES):
                        iv = idx_v[t, pl.ds(c * LANES, LANES)]
                        buf[tloc, c // 8, pl.ds((c % 8) * LANES, LANES)] = (
                            plsc.load_gather(row_v, [iv]))
                pltpu.async_copy(
                    buf, emb5.at[pl.ds(2 * gp, 2), trb, :, s], sem)
            return carry

        lax.fori_loop(0, TAU // 4, grp_body, 0)
        for bsel in range(2):
            gp = TAU // 2 - 2 + bsel
            pltpu.make_async_copy(
                bufs[bsel], emb5.at[pl.ds(2 * gp, 2), trb, :, s],
                sems[bsel]).wait()


_gather = functools.partial(
    pl.kernel,
    mesh=plsc.VectorSubcoreMesh(
        core_axis_name="c", subcore_axis_name="s",
        num_cores=NC, num_subcores=NS),
    out_type=jax.ShapeDtypeStruct((TAU, TRE, TCB, 8, 128), jnp.float32),
    scratch_types=[
        pltpu.VMEM((TAU, B), jnp.int32),
        pltpu.VMEM((V,), jnp.float32),
        pltpu.VMEM((2, 8, 128), jnp.float32),
        pltpu.VMEM((2, 8, 128), jnp.float32),
        pltpu.SemaphoreType.DMA,
        pltpu.SemaphoreType.DMA,
    ],
    compiler_params=pltpu.CompilerParams(
        use_tc_tiling_on_sc=True, needs_layout_passes=False,
        disable_bounds_checks=True),
)(_gather_body)


def _mm_body(h_ref, e_ref, w_ref, b_ref, o_ref):
    xh = h_ref[...].reshape(8, 8, 128).reshape(DM, 128)
    xe = e_ref[...].reshape(TAU, TRE, 8, 128).reshape(IN - DM, 128)
    xm = jnp.concatenate([xh, xe], axis=0)
    acc = jnp.dot(w_ref[0], xm, preferred_element_type=jnp.float32,
                  precision=lax.Precision.DEFAULT)
    o_ref[0] = acc + b_ref[0]


_matmul = pl.pallas_call(
    _mm_body,
    grid=(NT, TCB),
    in_specs=[
        pl.BlockSpec((1, 8, 1, 8, 128), lambda t, i: (NL - 1, 0, i, 0, 0)),
        pl.BlockSpec((TAU, TRE, 1, 8, 128), lambda t, i: (0, 0, i, 0, 0)),
        pl.BlockSpec((1, OUT, IN), lambda t, i: (t, 0, 0)),
        pl.BlockSpec((1, OUT, 1), lambda t, i: (t, 0, 0)),
    ],
    out_specs=pl.BlockSpec((1, OUT, 128), lambda t, i: (t, 0, i)),
    out_shape=jax.ShapeDtypeStruct((NT, OUT, B), jnp.float32),
)


def kernel(future, hidden, tables, W, b):
    fut = jnp.transpose(future.astype(jnp.int32), (2, 1, 0))
    tbl = jnp.transpose(tables, (0, 2, 1))
    h5 = (hidden.reshape(NL, B, 8, 8)
          .transpose(0, 2, 3, 1)
          .reshape(NL, 8, 8, 8, 128)
          .transpose(0, 1, 3, 2, 4))
    emb5 = _gather(fut, tbl)
    emb_out = emb5.transpose(2, 4, 0, 1, 3).reshape(B, TAU, FD)
    gc_t = _matmul(h5, emb5, jnp.transpose(W, (0, 2, 1)),
                   b.reshape(NT, OUT, 1))
    return emb_out, jnp.transpose(gc_t, (2, 0, 1))


# speedup vs baseline: 1.1153x; 1.1144x over previous
import functools

import jax
import jax.numpy as jnp
from jax import lax
from jax.experimental import pallas as pl
from jax.experimental.pallas import tpu as pltpu
from jax.experimental.pallas import tpu_sc as plsc

B = 1024
TAU = 20
F = 10
V = 100000
D = 16
DM = 64
NL = 2
NT = 2
IN = DM + TAU * D * F
OUT = (TAU + 1) * DM
FD = F * D

NC = 2
NS = 16
NW = NC * NS
PPW = FD // NW
LANES = 16

TRX = IN // 8
TRE = FD // 8
TCB = B // 128


def _gather_body(fut, tbl, emb5, idx_v, row_v, buf0, buf1, sem0, sem1):
    wid = lax.axis_index("s") * NC + lax.axis_index("c")
    bufs = (buf0, buf1)
    sems = (sem0, sem1)

    for k in range(PPW):
        p = PPW * wid + k
        f = p // D
        d = p % D
        trb = p // 8
        s = p % 8
        if k == 0:
            pltpu.sync_copy(fut.at[f], idx_v)
        else:
            @pl.when(f != (p - 1) // D)
            def _():
                pltpu.sync_copy(fut.at[f], idx_v)
        pltpu.sync_copy(tbl.at[f, d], row_v)

        def grp_body(g, carry):
            for bsel in range(2):
                gp = 2 * g + bsel
                buf, sem = bufs[bsel], sems[bsel]

                @pl.when(g > 0)
                def _():
                    pltpu.make_async_copy(
                        buf, emb5.at[pl.ds(2 * gp, 2), trb, :, s], sem).wait()

                for tloc in range(2):
                    t = 2 * gp + tloc
                    for c in range(B // LANES):
                        iv = idx_v[t, pl.ds(c * LANES, LANES)]
                        buf[tloc, c // 8, pl.ds((c % 8) * LANES, LANES)] = (
                            plsc.load_gather(row_v, [iv]))
                pltpu.async_copy(
                    buf, emb5.at[pl.ds(2 * gp, 2), trb, :, s], sem)
            return carry

        lax.fori_loop(0, TAU // 4, grp_body, 0)
        for bsel in range(2):
            gp = TAU // 2 - 2 + bsel
            pltpu.make_async_copy(
                bufs[bsel], emb5.at[pl.ds(2 * gp, 2), trb, :, s],
                sems[bsel]).wait()


_gather = functools.partial(
    pl.kernel,
    mesh=plsc.VectorSubcoreMesh(
        core_axis_name="c", subcore_axis_name="s",
        num_cores=NC, num_subcores=NS),
    out_type=jax.ShapeDtypeStruct((TAU, TRE, TCB, 8, 128), jnp.float32),
    scratch_types=[
        pltpu.VMEM((TAU, B), jnp.int32),
        pltpu.VMEM((V,), jnp.float32),
        pltpu.VMEM((2, 8, 128), jnp.float32),
        pltpu.VMEM((2, 8, 128), jnp.float32),
        pltpu.SemaphoreType.DMA,
        pltpu.SemaphoreType.DMA,
    ],
    compiler_params=pltpu.CompilerParams(
        use_tc_tiling_on_sc=True, needs_layout_passes=False,
        disable_bounds_checks=True),
)(_gather_body)


def _mm_body(h_ref, e_ref, w_ref, b_ref, o_ref):
    xh = h_ref[...].reshape(8, 2, 8, 128).transpose(0, 2, 1, 3).reshape(DM, 256)
    xe = (e_ref[...].reshape(TAU, TRE, 2, 8, 128)
          .transpose(0, 1, 3, 2, 4).reshape(IN - DM, 256))
    xm = jnp.concatenate([xh, xe], axis=0)
    acc = jnp.dot(w_ref[0], xm, preferred_element_type=jnp.float32,
                  precision=lax.Precision.DEFAULT)
    o_ref[0] = acc + b_ref[0]


_matmul = pl.pallas_call(
    _mm_body,
    grid=(NT, TCB // 2),
    in_specs=[
        pl.BlockSpec((1, 8, 2, 8, 128), lambda t, i: (NL - 1, 0, i, 0, 0)),
        pl.BlockSpec((TAU, TRE, 2, 8, 128), lambda t, i: (0, 0, i, 0, 0)),
        pl.BlockSpec((1, OUT, IN), lambda t, i: (t, 0, 0)),
        pl.BlockSpec((1, OUT, 1), lambda t, i: (t, 0, 0)),
    ],
    out_specs=pl.BlockSpec((1, OUT, 256), lambda t, i: (t, 0, i)),
    out_shape=jax.ShapeDtypeStruct((NT, OUT, B), jnp.float32),
)


def kernel(future, hidden, tables, W, b):
    fut = jnp.transpose(future.astype(jnp.int32), (2, 1, 0))
    tbl = jnp.transpose(tables, (0, 2, 1))
    h5 = (hidden.reshape(NL, B, 8, 8)
          .transpose(0, 2, 3, 1)
          .reshape(NL, 8, 8, 8, 128)
          .transpose(0, 1, 3, 2, 4))
    emb5 = _gather(fut, tbl)
    emb_out = emb5.transpose(2, 4, 0, 1, 3).reshape(B, TAU, FD)
    gc_t = _matmul(h5, emb5, jnp.transpose(W, (0, 2, 1)),
                   b.reshape(NT, OUT, 1))
    return emb_out, jnp.transpose(gc_t, (2, 0, 1))


# speedup vs baseline: 1.1250x; 1.0088x over previous
import functools

import jax
import jax.numpy as jnp
from jax import lax
from jax.experimental import pallas as pl
from jax.experimental.pallas import tpu as pltpu
from jax.experimental.pallas import tpu_sc as plsc

B = 1024
TAU = 20
F = 10
V = 100000
D = 16
DM = 64
NL = 2
NT = 2
IN = DM + TAU * D * F
OUT = (TAU + 1) * DM
FD = F * D

NC = 2
NS = 16
NW = NC * NS
PPW = FD // NW
LANES = 16

TRX = IN // 8
TRE = FD // 8
TCB = B // 128


def _gather_body(fut, tbl, emb5, idx_v, row_v, buf0, buf1, sem0, sem1):
    wid = lax.axis_index("s") * NC + lax.axis_index("c")
    bufs = (buf0, buf1)
    sems = (sem0, sem1)

    for k in range(PPW):
        p = PPW * wid + k
        f = p // D
        d = p % D
        trb = p // 8
        s = p % 8
        if k == 0:
            pltpu.sync_copy(fut.at[f], idx_v)
        else:
            @pl.when(f != (p - 1) // D)
            def _():
                pltpu.sync_copy(fut.at[f], idx_v)
        pltpu.sync_copy(tbl.at[f, d], row_v)

        def grp_body(g, carry):
            for bsel in range(2):
                gp = 2 * g + bsel
                buf, sem = bufs[bsel], sems[bsel]

                @pl.when(g > 0)
                def _():
                    pltpu.make_async_copy(
                        buf, emb5.at[pl.ds(2 * gp, 2), trb, :, s], sem).wait()

                for tloc in range(2):
                    t = 2 * gp + tloc
                    for c in range(B // LANES):
                        iv = idx_v[t, pl.ds(c * LANES, LANES)]
                        buf[tloc, c // 8, pl.ds((c % 8) * LANES, LANES)] = (
                            plsc.load_gather(row_v, [iv]))
                pltpu.async_copy(
                    buf, emb5.at[pl.ds(2 * gp, 2), trb, :, s], sem)
            return carry

        lax.fori_loop(0, TAU // 4, grp_body, 0)
        for bsel in range(2):
            gp = TAU // 2 - 2 + bsel
            pltpu.make_async_copy(
                bufs[bsel], emb5.at[pl.ds(2 * gp, 2), trb, :, s],
                sems[bsel]).wait()


_gather = functools.partial(
    pl.kernel,
    mesh=plsc.VectorSubcoreMesh(
        core_axis_name="c", subcore_axis_name="s",
        num_cores=NC, num_subcores=NS),
    out_type=jax.ShapeDtypeStruct((TAU, TRE, TCB, 8, 128), jnp.float32),
    scratch_types=[
        pltpu.VMEM((TAU, B), jnp.int32),
        pltpu.VMEM((V,), jnp.float32),
        pltpu.VMEM((2, 8, 128), jnp.float32),
        pltpu.VMEM((2, 8, 128), jnp.float32),
        pltpu.SemaphoreType.DMA,
        pltpu.SemaphoreType.DMA,
    ],
    compiler_params=pltpu.CompilerParams(
        use_tc_tiling_on_sc=True, needs_layout_passes=False,
        disable_bounds_checks=True),
)(_gather_body)


def _mm_body(h_ref, e_ref, w_ref, b_ref, o_ref):
    xh = h_ref[...].reshape(8, 4, 8, 128).transpose(0, 2, 1, 3).reshape(DM, 512)
    xe = (e_ref[...].reshape(TAU, TRE, 4, 8, 128)
          .transpose(0, 1, 3, 2, 4).reshape(IN - DM, 512))
    xm = jnp.concatenate([xh, xe], axis=0)
    acc = jnp.dot(w_ref[0], xm, preferred_element_type=jnp.float32,
                  precision=lax.Precision.DEFAULT)
    o_ref[0] = acc + b_ref[0]


_matmul = pl.pallas_call(
    _mm_body,
    grid=(NT, TCB // 4),
    in_specs=[
        pl.BlockSpec((1, 8, 4, 8, 128), lambda t, i: (NL - 1, 0, i, 0, 0)),
        pl.BlockSpec((TAU, TRE, 4, 8, 128), lambda t, i: (0, 0, i, 0, 0)),
        pl.BlockSpec((1, OUT, IN), lambda t, i: (t, 0, 0)),
        pl.BlockSpec((1, OUT, 1), lambda t, i: (t, 0, 0)),
    ],
    out_specs=pl.BlockSpec((1, OUT, 512), lambda t, i: (t, 0, i)),
    out_shape=jax.ShapeDtypeStruct((NT, OUT, B), jnp.float32),
)


def kernel(future, hidden, tables, W, b):
    fut = jnp.transpose(future.astype(jnp.int32), (2, 1, 0))
    tbl = jnp.transpose(tables, (0, 2, 1))
    h5 = (hidden.reshape(NL, B, 8, 8)
          .transpose(0, 2, 3, 1)
          .reshape(NL, 8, 8, 8, 128)
          .transpose(0, 1, 3, 2, 4))
    emb5 = _gather(fut, tbl)
    emb_out = emb5.transpose(2, 4, 0, 1, 3).reshape(B, TAU, FD)
    gc_t = _matmul(h5, emb5, jnp.transpose(W, (0, 2, 1)),
                   b.reshape(NT, OUT, 1))
    return emb_out, jnp.transpose(gc_t, (2, 0, 1))


# speedup vs baseline: 1.1367x; 1.0104x over previous
import functools

import jax
import jax.numpy as jnp
from jax import lax
from jax.experimental import pallas as pl
from jax.experimental.pallas import tpu as pltpu
from jax.experimental.pallas import tpu_sc as plsc

B = 1024
TAU = 20
F = 10
V = 100000
D = 16
DM = 64
NL = 2
NT = 2
IN = DM + TAU * D * F
OUT = (TAU + 1) * DM
FD = F * D

NC = 2
NS = 16
NW = NC * NS
PPW = FD // NW
LANES = 16

TRX = IN // 8
TRE = FD // 8
TCB = B // 128


def _gather_body(fut, tbl, emb5, idx_v, row_v, buf0, buf1, sem0, sem1):
    wid = lax.axis_index("s") * NC + lax.axis_index("c")
    bufs = (buf0, buf1)
    sems = (sem0, sem1)

    for k in range(PPW):
        p = PPW * wid + k
        f = p // D
        d = p % D
        trb = p // 8
        s = p % 8
        if k == 0:
            pltpu.sync_copy(fut.at[f], idx_v)
        else:
            @pl.when(f != (p - 1) // D)
            def _():
                pltpu.sync_copy(fut.at[f], idx_v)
        pltpu.sync_copy(tbl.at[f, d], row_v)

        def grp_body(g, carry):
            for bsel in range(2):
                gp = 2 * g + bsel
                buf, sem = bufs[bsel], sems[bsel]

                @pl.when(g > 0)
                def _():
                    pltpu.make_async_copy(
                        buf, emb5.at[pl.ds(2 * gp, 2), trb, :, s], sem).wait()

                for tloc in range(2):
                    t = 2 * gp + tloc
                    for c in range(B // LANES):
                        iv = idx_v[t, pl.ds(c * LANES, LANES)]
                        buf[tloc, c // 8, pl.ds((c % 8) * LANES, LANES)] = (
                            plsc.load_gather(row_v, [iv]))
                pltpu.async_copy(
                    buf, emb5.at[pl.ds(2 * gp, 2), trb, :, s], sem)
            return carry

        lax.fori_loop(0, TAU // 4, grp_body, 0)
        for bsel in range(2):
            gp = TAU // 2 - 2 + bsel
            pltpu.make_async_copy(
                bufs[bsel], emb5.at[pl.ds(2 * gp, 2), trb, :, s],
                sems[bsel]).wait()


_gather = functools.partial(
    pl.kernel,
    mesh=plsc.VectorSubcoreMesh(
        core_axis_name="c", subcore_axis_name="s",
        num_cores=NC, num_subcores=NS),
    out_type=jax.ShapeDtypeStruct((TAU, TRE, TCB, 8, 128), jnp.float32),
    scratch_types=[
        pltpu.VMEM((TAU, B), jnp.int32),
        pltpu.VMEM((V,), jnp.float32),
        pltpu.VMEM((2, 8, 128), jnp.float32),
        pltpu.VMEM((2, 8, 128), jnp.float32),
        pltpu.SemaphoreType.DMA,
        pltpu.SemaphoreType.DMA,
    ],
    compiler_params=pltpu.CompilerParams(
        use_tc_tiling_on_sc=True, needs_layout_passes=False,
        disable_bounds_checks=True),
)(_gather_body)


MB = OUT // 2


def _mm_body(h_ref, e_ref, w_ref, b_ref, o_ref):
    xh = h_ref[...].reshape(8, 8, 8, 128).transpose(0, 2, 1, 3).reshape(DM, B)
    xe = (e_ref[...].reshape(TAU, TRE, 8, 8, 128)
          .transpose(0, 1, 3, 2, 4).reshape(IN - DM, B))
    xm = jnp.concatenate([xh, xe], axis=0)
    acc = jnp.dot(w_ref[0], xm, preferred_element_type=jnp.float32,
                  precision=lax.Precision.DEFAULT)
    o_ref[0] = acc + b_ref[0]


_matmul = pl.pallas_call(
    _mm_body,
    grid=(NT, OUT // MB),
    in_specs=[
        pl.BlockSpec((1, 8, 8, 8, 128), lambda t, i: (NL - 1, 0, 0, 0, 0)),
        pl.BlockSpec((TAU, TRE, 8, 8, 128), lambda t, i: (0, 0, 0, 0, 0)),
        pl.BlockSpec((1, MB, IN), lambda t, i: (t, i, 0)),
        pl.BlockSpec((1, MB, 1), lambda t, i: (t, i, 0)),
    ],
    out_specs=pl.BlockSpec((1, MB, B), lambda t, i: (t, i, 0)),
    out_shape=jax.ShapeDtypeStruct((NT, OUT, B), jnp.float32),
)


def kernel(future, hidden, tables, W, b):
    fut = jnp.transpose(future.astype(jnp.int32), (2, 1, 0))
    tbl = jnp.transpose(tables, (0, 2, 1))
    h5 = (hidden.reshape(NL, B, 8, 8)
          .transpose(0, 2, 3, 1)
          .reshape(NL, 8, 8, 8, 128)
          .transpose(0, 1, 3, 2, 4))
    emb5 = _gather(fut, tbl)
    emb_out = emb5.transpose(2, 4, 0, 1, 3).reshape(B, TAU, FD)
    gc_t = _matmul(h5, emb5, jnp.transpose(W, (0, 2, 1)),
                   b.reshape(NT, OUT, 1))
    return emb_out, jnp.transpose(gc_t, (2, 0, 1))
